# baseline (speedup 1.0000x reference)
import functools

import jax
import jax.numpy as jnp
from jax import lax
from jax.experimental import pallas as pl
from jax.experimental.pallas import tpu as pltpu, tpu_sc as plsc

_BATCH = 4
_SEQ = 8192
_D = 64

_NC = 2
_NS = 16
_NW = _NC * _NS

_NDC = 8
_DC = _D // _NDC
_NLQ = _NW // _NDC
_LQ = _SEQ // _NLQ
_NV = (_DC * _LQ) // 16


def _pos_add_body(x_hbm, tab_hbm, out_hbm, buf0, buf1, buf2, buf3, buft,
                  semt, sx0, sx1, sx2, sx3, so0, so1, so2, so3):
    wid = lax.axis_index("s") * _NC + lax.axis_index("c")
    dc0 = (wid // _NLQ) * _DC
    l0 = (wid % _NLQ) * _LQ

    bufs = (buf0, buf1, buf2, buf3)
    sx = (sx0, sx1, sx2, sx3)
    so = (so0, so1, so2, so3)

    def xsl(b):
        return x_hbm.at[b, pl.ds(dc0, _DC), pl.ds(l0, _LQ)]

    def osl(b):
        return out_hbm.at[b, pl.ds(dc0, _DC), pl.ds(l0, _LQ)]

    def add_into(cur):
        @plsc.parallel_loop(0, _NV, unroll=16)
        def _add(i):
            r = i // (_LQ // 16)
            s = pl.ds((i % (_LQ // 16)) * 16, 16)
            plsc.addupdate(cur.at[r, s], buft[r, s])

    ct = pltpu.async_copy(tab_hbm.at[pl.ds(dc0, _DC), pl.ds(l0, _LQ)],
                          buft, semt)
    for b in range(_BATCH):
        pltpu.async_copy(xsl(b), bufs[b], sx[b])
    ct.wait()

    for b in range(_BATCH):
        pltpu.make_async_copy(xsl(b), bufs[b], sx[b]).wait()
        add_into(bufs[b])
        pltpu.async_copy(bufs[b], osl(b), so[b])

    for b in range(_BATCH):
        pltpu.make_async_copy(bufs[b], osl(b), so[b]).wait()


def _make_pos_add(interpret=False):
    return functools.partial(
        pl.kernel,
        out_type=jax.ShapeDtypeStruct((_BATCH, _D, _SEQ), jnp.float32),
        mesh=plsc.VectorSubcoreMesh(core_axis_name="c", subcore_axis_name="s"),
        scratch_types=(
            [pltpu.VMEM((_DC, _LQ), jnp.float32)] * 5
            + [pltpu.SemaphoreType.DMA] * 9
        ),
        interpret=interpret,
    )(_pos_add_body)


_pos_add = _make_pos_add()


def kernel(x, pos_emb_table):
    xt = jnp.transpose(x, (0, 2, 1))
    tabt = jnp.transpose(pos_emb_table)
    outt = _pos_add(xt, tabt)
    return jnp.transpose(outt, (0, 2, 1))


# speedup vs baseline: 1.0346x; 1.0346x over previous
import functools

import jax
import jax.numpy as jnp
from jax import lax
from jax.experimental import pallas as pl
from jax.experimental.pallas import tpu as pltpu, tpu_sc as plsc

_BATCH = 4
_SEQ = 8192
_D = 64

_NC = 2
_NS = 16
_NW = _NC * _NS

_NDC = 8
_DC = _D // _NDC
_NLQ = _NW // _NDC
_LQ = _SEQ // _NLQ
_NV = (_DC * _LQ) // 16


def _pos_add_body(x_hbm, tab_hbm, out_hbm, buf0, buf1, buf2, buf3, buft,
                  semt, sx0, sx1, sx2, sx3, so0, so1, so2, so3):
    wid = lax.axis_index("s") * _NC + lax.axis_index("c")
    dc0 = (wid // _NLQ) * _DC
    l0 = (wid % _NLQ) * _LQ

    bufs = (buf0, buf1, buf2, buf3)
    sx = (sx0, sx1, sx2, sx3)
    so = (so0, so1, so2, so3)

    def xsl(b):
        return x_hbm.at[b, pl.ds(dc0, _DC), pl.ds(l0, _LQ)]

    def osl(b):
        return out_hbm.at[b, pl.ds(dc0, _DC), pl.ds(l0, _LQ)]

    _HV = _NV // 2
    _HL = _LQ // 2

    def add_half(cur, h):
        @plsc.parallel_loop(h * _HV, (h + 1) * _HV, unroll=16)
        def _add(i):
            r = i // (_LQ // 16)
            s = pl.ds((i % (_LQ // 16)) * 16, 16)
            plsc.addupdate(cur.at[r, s], buft[r, s])

    def out_half(cur, b, h):
        pltpu.async_copy(
            cur.at[pl.ds(0, _DC), pl.ds(h * _HL, _HL)],
            out_hbm.at[b, pl.ds(dc0, _DC), pl.ds(l0 + h * _HL, _HL)],
            so[b])

    def process(cur, b):
        for h in range(2):
            add_half(cur, h)
            out_half(cur, b, h)

    def drain_out(cur, b):
        pltpu.make_async_copy(cur, osl(b), so[b]).wait()

    ct = pltpu.async_copy(tab_hbm.at[pl.ds(dc0, _DC), pl.ds(l0, _LQ)],
                          buft, semt)
    pltpu.async_copy(xsl(0), bufs[0], sx[0])
    pltpu.async_copy(xsl(1), bufs[1], sx[1])
    ct.wait()

    pltpu.make_async_copy(xsl(0), bufs[0], sx[0]).wait()
    pltpu.async_copy(xsl(2), bufs[2], sx[2])
    process(bufs[0], 0)
    pltpu.make_async_copy(xsl(1), bufs[1], sx[1]).wait()
    pltpu.async_copy(xsl(3), bufs[3], sx[3])
    process(bufs[1], 1)
    pltpu.make_async_copy(xsl(2), bufs[2], sx[2]).wait()
    process(bufs[2], 2)
    pltpu.make_async_copy(xsl(3), bufs[3], sx[3]).wait()
    process(bufs[3], 3)

    for b in range(_BATCH):
        drain_out(bufs[b], b)


def _make_pos_add(interpret=False):
    return functools.partial(
        pl.kernel,
        out_type=jax.ShapeDtypeStruct((_BATCH, _D, _SEQ), jnp.float32),
        mesh=plsc.VectorSubcoreMesh(core_axis_name="c", subcore_axis_name="s"),
        scratch_types=(
            [pltpu.VMEM((_DC, _LQ), jnp.float32)] * 5
            + [pltpu.SemaphoreType.DMA] * 9
        ),
        interpret=interpret,
    )(_pos_add_body)


_pos_add = _make_pos_add()


def kernel(x, pos_emb_table):
    xt = jnp.transpose(x, (0, 2, 1))
    tabt = jnp.transpose(pos_emb_table)
    outt = _pos_add(xt, tabt)
    return jnp.transpose(outt, (0, 2, 1))
